# baseline (speedup 1.0000x reference)
import functools

import jax
import jax.numpy as jnp
from jax import lax
from jax.experimental import pallas as pl
from jax.experimental.pallas import tpu as pltpu
from jax.experimental.pallas import tpu_sc as plsc

_L = 2048
_B = 4
_LIMIT = 256
_NROWS = _B * _L
_NW = 32
_GR = 32
_NG = _NROWS // _GR
_GPW = _NG // _NW
_WIN = _LIMIT + 128
_BUF = 640
_AMAX = _L - _WIN
_PLAIN_MAX = _L - _LIMIT - _GR
_NBUF = 2


_UNROLL = 8


def _row_chunks(win, t, rbase, lane):
    s = t & 15
    base = pl.multiple_of(rbase + (t - s), 16)
    rot = (lane + s) & 15
    seam = lane < (16 - s)
    rc = jnp.take(win[t, pl.ds(base, 16)], rot, mode="wrap")
    for k in range(_LIMIT // 16):
        cn = win[t, pl.ds(base + 16 * (k + 1), 16)]
        rcn = jnp.take(cn, rot, mode="wrap")
        yield k, jnp.where(seam, rc, rcn)
        rc = rcn


def _emit_masked(win, obuf, orow, ig, rbase, lane):
    lcol = lane + ig

    @plsc.parallel_loop(0, _GR, unroll=_UNROLL)
    def _row(t):
        for k, v in _row_chunks(win, t, rbase, lane):
            v = jnp.where(lcol < (_L - t - 16 * k), v, 0.0)
            obuf[orow + t, pl.ds(16 * k, 16)] = v


def _sc_body(x_ref, out_ref, *scr):
    wins = scr[:_NBUF]
    obuf = scr[_NBUF]
    sems = scr[_NBUF + 1:_NBUF + 1 + _NBUF]
    osem = scr[_NBUF + 1 + _NBUF]
    wid = lax.axis_index("s") * 2 + lax.axis_index("c")
    lane = lax.iota(jnp.int32, 16)

    def _grp(j):
        g = wid + _NW * j
        G = pl.multiple_of(g * _GR, _GR)
        ig = lax.rem(G, _L)
        a = pl.multiple_of(jnp.minimum(ig - lax.rem(ig, 128), _AMAX), 128)
        return G, ig, a

    def _issue(j, p):
        G, _, a = _grp(jnp.minimum(j, _GPW - 1))
        pltpu.async_copy(
            x_ref.at[pl.ds(G, _GR), pl.ds(a, _WIN)],
            wins[p].at[:, pl.ds(0, _WIN)],
            sems[p],
        )

    def _wait(p):
        pltpu.make_async_copy(
            x_ref.at[pl.ds(0, _GR), pl.ds(0, _WIN)],
            wins[p].at[:, pl.ds(0, _WIN)],
            sems[p],
        ).wait()

    def _out_copy(j):
        G = pl.multiple_of((wid + _NW * j) * _GR, _GR)
        return pltpu.make_async_copy(
            obuf.at[pl.ds(j * _GR, _GR), :],
            out_ref.at[pl.ds(G, _GR), :],
            osem,
        )

    for p in range(_NBUF):
        _issue(jnp.int32(p), p)

    def _step(it, carry):
        for p in range(_NBUF):
            j = it * _NBUF + p
            _wait(p)
            _, ig, a = _grp(j)
            rbase = pl.multiple_of(ig - a, 16)
            orow = j * _GR
            _emit_masked(wins[p], obuf, orow, ig, rbase, lane)
            _issue(j + _NBUF, p)
            _out_copy(j).start()
        return carry

    lax.fori_loop(0, _GPW // _NBUF, _step, 0, unroll=False)

    for p in range(_NBUF):
        _wait(p)
    for j in range(_GPW):
        _out_copy(jnp.int32(j)).wait()


@jax.jit
def _run(x2d):
    call = pl.kernel(
        _sc_body,
        out_type=jax.ShapeDtypeStruct((_NROWS, _LIMIT), jnp.float32),
        mesh=plsc.VectorSubcoreMesh(core_axis_name="c", subcore_axis_name="s"),
        scratch_types=(
            [pltpu.VMEM((_GR, _BUF), jnp.float32) for _ in range(_NBUF)]
            + [pltpu.VMEM((_GPW * _GR, _LIMIT), jnp.float32)]
            + [pltpu.SemaphoreType.DMA for _ in range(_NBUF)]
            + [pltpu.SemaphoreType.DMA]
        ),
    )
    return call(x2d)


def kernel(x):
    B, L, D = x.shape
    out = _run(x.reshape(B * L, D))
    return out.reshape(B, L, _LIMIT)


# speedup vs baseline: 1.0015x; 1.0015x over previous
import jax
import jax.numpy as jnp
from jax import lax
from jax.experimental import pallas as pl
from jax.experimental.pallas import tpu as pltpu
from jax.experimental.pallas import tpu_sc as plsc

_L = 2048
_B = 4
_LIMIT = 256
_NROWS = _B * _L
_NW = 32
_GR = 32
_NG = _NROWS // _GR
_GPW = _NG // _NW
_WIN = _LIMIT + 128
_BUF = 640
_AMAX = _L - _WIN
_NBUF = 2
_UNROLL = 8


def _row_chunks(win, t, rbase, lane):
    s = t & 15
    base = pl.multiple_of(rbase + (t - s), 16)
    rot = (lane + s) & 15
    seam = lane < (16 - s)
    rc = jnp.take(win[t, pl.ds(base, 16)], rot, mode="wrap")
    for k in range(_LIMIT // 16):
        cn = win[t, pl.ds(base + 16 * (k + 1), 16)]
        rcn = jnp.take(cn, rot, mode="wrap")
        yield k, jnp.where(seam, rc, rcn)
        rc = rcn


def _emit_masked(win, obuf, orow, ig, rbase, lane):
    lcol = lane + ig

    @plsc.parallel_loop(0, _GR, unroll=_UNROLL)
    def _row(t):
        for k, v in _row_chunks(win, t, rbase, lane):
            v = jnp.where(lcol < (_L - t - 16 * k), v, 0.0)
            obuf[orow + t, pl.ds(16 * k, 16)] = v


def _sc_body(x_ref, out_ref, *scr):
    wins = scr[:_NBUF]
    obuf = scr[_NBUF]
    sems = scr[_NBUF + 1:_NBUF + 1 + _NBUF]
    osem = scr[_NBUF + 1 + _NBUF]
    wid = lax.axis_index("s") * 2 + lax.axis_index("c")
    lane = lax.iota(jnp.int32, 16)

    def _grp(j):
        g = wid + _NW * j
        G = pl.multiple_of(g * _GR, _GR)
        ig = lax.rem(G, _L)
        a = pl.multiple_of(jnp.minimum(ig - lax.rem(ig, 128), _AMAX), 128)
        return G, ig, a

    def _issue(j, p):
        G, _, a = _grp(jnp.minimum(j, _GPW - 1))
        pltpu.async_copy(
            x_ref.at[pl.ds(G, _GR), pl.ds(a, _WIN)],
            wins[p].at[:, pl.ds(0, _WIN)],
            sems[p],
        )

    def _wait(p):
        pltpu.make_async_copy(
            x_ref.at[pl.ds(0, _GR), pl.ds(0, _WIN)],
            wins[p].at[:, pl.ds(0, _WIN)],
            sems[p],
        ).wait()

    def _out_copy(j):
        G = pl.multiple_of((wid + _NW * j) * _GR, _GR)
        return pltpu.make_async_copy(
            obuf.at[pl.ds(j * _GR, _GR), :],
            out_ref.at[pl.ds(G, _GR), :],
            osem,
        )

    for p in range(_NBUF):
        _issue(jnp.int32(p), p)

    def _step(it, carry):
        for p in range(_NBUF):
            j = it * _NBUF + p
            _wait(p)
            _, ig, a = _grp(j)
            rbase = pl.multiple_of(ig - a, 16)
            orow = j * _GR
            _emit_masked(wins[p], obuf, orow, ig, rbase, lane)
            _issue(j + _NBUF, p)
            _out_copy(j).start()
        return carry

    lax.fori_loop(0, _GPW // _NBUF, _step, 0, unroll=False)

    for p in range(_NBUF):
        _wait(p)
    for j in range(_GPW):
        _out_copy(jnp.int32(j)).wait()


@jax.jit
def _run(x2d):
    call = pl.kernel(
        _sc_body,
        out_type=jax.ShapeDtypeStruct((_NROWS, _LIMIT), jnp.float32),
        mesh=plsc.VectorSubcoreMesh(core_axis_name="c", subcore_axis_name="s"),
        scratch_types=(
            [pltpu.VMEM((_GR, _BUF), jnp.float32) for _ in range(_NBUF)]
            + [pltpu.VMEM((_GPW * _GR, _LIMIT), jnp.float32)]
            + [pltpu.SemaphoreType.DMA for _ in range(_NBUF)]
            + [pltpu.SemaphoreType.DMA]
        ),
    )
    return call(x2d)


def kernel(x):
    B, L, D = x.shape
    out = _run(x.reshape(B * L, D))
    return out.reshape(B, L, _LIMIT)
